# baseline (speedup 1.0000x reference)
import functools

import jax
import jax.numpy as jnp
from jax import lax
from jax.experimental import pallas as pl
from jax.experimental.pallas import tpu as pltpu
from jax.experimental.pallas import tpu_sc as plsc

N, E, F = 10000, 160000, 128
FC = 32
NC = F // FC
NS = 16
EB = 80
BPT = E // EB // NS
RB = 80
NSLAB = N // RB


def _unit_body(r_ref, u_ref):
    r = r_ref[...]
    n2 = jnp.sum(r * r, axis=1, keepdims=True)
    u_ref[...] = jnp.transpose(r / jnp.sqrt(n2 + 1e-12))


def _unit_vectors(r):
    ebk = 3200
    return pl.pallas_call(
        _unit_body,
        grid=(E // ebk,),
        in_specs=[pl.BlockSpec((ebk, 3), lambda i: (i, 0))],
        out_specs=pl.BlockSpec((3, ebk), lambda i: (0, i)),
        out_shape=jax.ShapeDtypeStruct((3, E), jnp.float32),
    )(r)


def _sc_body(idx_hbm, u_hbm, f_hbm, out0, out1, out2, out3,
             acc, idx0_v, idx1_v, idx2_v, u0_v, u1_v, u2_v,
             vals0_v, vals1_v, vals2_v, st_v,
             dsem0, dsem1, dsem2, ssem0, ssem1, ssem2):
    outs = (out0, out1, out2, out3)
    idx_bufs = (idx0_v, idx1_v, idx2_v)
    u_bufs = (u0_v, u1_v, u2_v)
    vals_bufs = (vals0_v, vals1_v, vals2_v)
    dsems = (dsem0, dsem1, dsem2)
    ssems = (ssem0, ssem1, ssem2)
    c = lax.axis_index("c")
    s = lax.axis_index("s")
    n_my_slabs = (NSLAB // NS) + jnp.where(s < NSLAB % NS, 1, 0)

    zero16 = jnp.zeros((16,), jnp.float32)

    def start_dma(b, p):
        e0 = (s + b * NS) * EB
        pltpu.async_copy(idx_hbm.at[pl.ds(e0, EB)], idx_bufs[p], dsems[p])
        for d in range(3):
            pltpu.async_copy(u_hbm.at[pl.ds(d * E + e0, EB)],
                             u_bufs[p].at[pl.ds(d * EB, EB)], dsems[p])
        pltpu.async_copy(f_hbm.at[pl.ds(e0, EB), :], vals_bufs[p], dsems[p])

    def wait_dma(p):
        pltpu.make_async_copy(idx_hbm.at[pl.ds(0, EB)], idx_bufs[p], dsems[p]).wait()
        for d in range(3):
            pltpu.make_async_copy(u_hbm.at[pl.ds(0, EB)],
                                  u_bufs[p].at[pl.ds(0, EB)], dsems[p]).wait()
        pltpu.make_async_copy(f_hbm.at[pl.ds(0, EB), :], vals_bufs[p], dsems[p]).wait()

    def start_scatter(p):
        pltpu.async_copy(vals_bufs[p], acc.at[idx_bufs[p]], ssems[p], add=True)

    def wait_scatter(p):
        pltpu.make_async_copy(vals_bufs[p], acc.at[idx_bufs[p]], ssems[p]).wait()

    def _pack_su(n_rows, s_off, t_offs):
        def sq_body(i, carry):
            for gg in range(FC // 16):
                x0 = vals0_v[i, pl.ds(t_offs[0] + gg * 16, 16)]
                x1 = vals0_v[i, pl.ds(t_offs[1] + gg * 16, 16)]
                x2 = vals0_v[i, pl.ds(t_offs[2] + gg * 16, 16)]
                sv = vals0_v[i, pl.ds(s_off + gg * 16, 16)]
                st_v[pl.ds(i * 64 + gg * 16, 16)] = sv
                st_v[pl.ds(i * 64 + 32 + gg * 16, 16)] = x0 * x0 + x1 * x1 + x2 * x2
            return carry

        lax.fori_loop(0, n_rows, sq_body, 0)

    for k in range(2):
        cidx = c * 2 + k
        col0 = pl.multiple_of(cidx * FC, FC)

        def compute_block(p):
            vals_v = vals_bufs[p]
            u_v = u_bufs[p]

            for core_id in range(2):
                cidx_s = core_id * 2 + k
                cs = cidx_s * FC
                ts = tuple(FC * j + (FC if j >= cidx_s else 0)
                           for j in range(3))

                @pl.when(c == core_id)
                def _(cs=cs, ts=ts):
                    def group_body(g, gcarry):
                        base = pl.multiple_of(g * 16, 16)
                        ur0 = u_v[pl.ds(base, 16)]
                        ur1 = u_v[pl.ds(EB + base, 16)]
                        ur2 = u_v[pl.ds(2 * EB + base, 16)]
                        for lane in range(16):
                            e = base + lane
                            u0 = ur0[lane]
                            u1 = ur1[lane]
                            u2 = ur2[lane]
                            for gg in range(FC // 16):
                                fr = vals_v[e, pl.ds(cs + gg * 16, 16)]
                                vals_v[e, pl.ds(ts[0] + gg * 16, 16)] = u0 * fr
                                vals_v[e, pl.ds(ts[1] + gg * 16, 16)] = u1 * fr
                                vals_v[e, pl.ds(ts[2] + gg * 16, 16)] = u2 * fr
                        return gcarry

                    lax.fori_loop(0, EB // 16, group_body, 0)

        def zfill(i, carry):
            for gg in range(F // 16):
                vals0_v[i, pl.ds(gg * 16, 16)] = zero16
            return carry

        lax.fori_loop(0, RB, zfill, 0)

        def zslab(i, carry):
            r0 = (s + i * NS) * RB
            pltpu.sync_copy(vals0_v, acc.at[pl.ds(r0, RB), :])
            return carry

        lax.fori_loop(0, n_my_slabs, zslab, 0)
        plsc.subcore_barrier()

        def slot(b, j):
            p = j % 3
            wait_dma(p)
            compute_block(p)

            @pl.when(b > 0)
            def _():
                wait_scatter((j + 2) % 3)

            start_scatter(p)

            @pl.when(b + 2 < BPT)
            def _():
                start_dma(b + 2, (j + 2) % 3)

        start_dma(0, 0)
        start_dma(1, 1)

        def tri_body(i, carry):
            for j in range(3):
                slot(3 * i + j, j)
            return carry

        lax.fori_loop(0, BPT // 3, tri_body, 0)
        for j in range(BPT - 3 * (BPT // 3)):
            slot(3 * (BPT // 3) + j, j)
        wait_scatter((BPT - 1) % 3)
        plsc.subcore_barrier()

        for core_id in range(2):
            out_ref = outs[core_id * 2 + k]
            cidx_s = core_id * 2 + k
            cs = cidx_s * FC
            ts = tuple(FC * j + (FC if j >= cidx_s else 0) for j in range(3))

            @pl.when(c == core_id)
            def _(cs=cs, ts=ts, out_ref=out_ref):
                def fslab(i, carry):
                    r0 = (s + i * NS) * RB
                    pltpu.sync_copy(acc.at[pl.ds(r0, RB), :], vals0_v)
                    _pack_su(RB, cs, ts)
                    pltpu.sync_copy(st_v, out_ref.at[pl.ds(r0 * 64, RB * 64)])
                    return carry

                lax.fori_loop(0, n_my_slabs, fslab, 0)

        plsc.subcore_barrier()


def _segment_sums(idx_j, u_t, f_ij):
    mesh = plsc.VectorSubcoreMesh(core_axis_name="c", subcore_axis_name="s")
    fn = functools.partial(
        pl.kernel,
        out_type=[jax.ShapeDtypeStruct((N * 64,), jnp.float32)] * 4,
        mesh=mesh,
        scratch_types=[
            pltpu.VMEM_SHARED((N, F), jnp.float32),
            pltpu.VMEM((EB,), jnp.int32),
            pltpu.VMEM((EB,), jnp.int32),
            pltpu.VMEM((EB,), jnp.int32),
            pltpu.VMEM((3 * EB,), jnp.float32),
            pltpu.VMEM((3 * EB,), jnp.float32),
            pltpu.VMEM((3 * EB,), jnp.float32),
            pltpu.VMEM((EB, F), jnp.float32),
            pltpu.VMEM((EB, F), jnp.float32),
            pltpu.VMEM((EB, F), jnp.float32),
            pltpu.VMEM((RB * 64,), jnp.float32),
            pltpu.SemaphoreType.DMA,
            pltpu.SemaphoreType.DMA,
            pltpu.SemaphoreType.DMA,
            pltpu.SemaphoreType.DMA,
            pltpu.SemaphoreType.DMA,
            pltpu.SemaphoreType.DMA,
        ],
    )(_sc_body)
    return [o.reshape(N, 64) for o in fn(idx_j, u_t.reshape(3 * E), f_ij)]


def _gelu(x):
    return jax.nn.gelu(x)


def _dense_body(emb_ref, pc_ref, o0_ref, o1_ref, o2_ref, o3_ref,
                wemb_ref, bemb_ref, wchg_ref, bchg_ref,
                w1_ref, b1_ref, w2_ref, b2_ref,
                w3a_ref, b3a_ref, w4a_ref, b4a_ref,
                w3q_ref, b3q_ref, w4qt_ref, b4q_ref,
                da_ref, dq_ref):
    emb = emb_ref[...]
    a_t = jnp.dot(emb, wemb_ref[...], preferred_element_type=jnp.float32) + bemb_ref[...]
    q_t = pc_ref[...] * jnp.sum(wchg_ref[...], axis=0, keepdims=True) + bchg_ref[...]

    o_blocks = [o_ref[...] for o_ref in (o0_ref, o1_ref, o2_ref, o3_ref)]
    s_v = jnp.concatenate([o[:, :FC] for o in o_blocks], axis=1)
    uu = jnp.concatenate([o[:, FC:] for o in o_blocks], axis=1)
    radial_a = a_t * s_v
    vector_a = jnp.sqrt(a_t * a_t * uu + 1e-12)
    radial_q = q_t * s_v
    vector_q = jnp.sqrt(q_t * q_t * uu + 1e-12)
    comb = jnp.concatenate([radial_a, vector_a, radial_q, vector_q], axis=1)

    h = _gelu(jnp.dot(comb, w1_ref[...], preferred_element_type=jnp.float32) + b1_ref[...])
    h = _gelu(jnp.dot(h, w2_ref[...], preferred_element_type=jnp.float32) + b2_ref[...])
    ha = _gelu(jnp.dot(h, w3a_ref[...], preferred_element_type=jnp.float32) + b3a_ref[...])
    da_ref[...] = jnp.dot(ha, w4a_ref[...], preferred_element_type=jnp.float32) + b4a_ref[...]
    hq = _gelu(jnp.dot(h, w3q_ref[...], preferred_element_type=jnp.float32) + b3q_ref[...])
    dq_ref[...] = jnp.sum(hq * w4qt_ref[...], axis=1, keepdims=True) + b4q_ref[...]


def _dense_stage(emb, pc, su_chunks, wemb, bemb, wchg, bchg,
                 w1, b1, w2, b2, w3a, b3a, w4a, b4a, w3q, b3q, w4qt, b4q):
    nb = 1000
    grid = N // nb
    row_spec = lambda shape: pl.BlockSpec((nb,) + shape[1:], lambda i: (i,) + (0,) * (len(shape) - 1))
    full_spec = lambda shape: pl.BlockSpec(shape, lambda i: (0,) * len(shape))
    in_specs = [
        row_spec((N, F)), row_spec((N, 1)),
        row_spec((N, 64)), row_spec((N, 64)), row_spec((N, 64)), row_spec((N, 64)),
        full_spec(wemb.shape), full_spec(bemb.shape),
        full_spec(wchg.shape), full_spec(bchg.shape),
        full_spec(w1.shape), full_spec(b1.shape),
        full_spec(w2.shape), full_spec(b2.shape),
        full_spec(w3a.shape), full_spec(b3a.shape),
        full_spec(w4a.shape), full_spec(b4a.shape),
        full_spec(w3q.shape), full_spec(b3q.shape),
        full_spec(w4qt.shape), full_spec(b4q.shape),
    ]
    return pl.pallas_call(
        _dense_body,
        grid=(grid,),
        in_specs=in_specs,
        out_specs=[
            pl.BlockSpec((nb, F), lambda i: (i, 0)),
            pl.BlockSpec((nb, 1), lambda i: (i, 0)),
        ],
        out_shape=[
            jax.ShapeDtypeStruct((N, F), jnp.float32),
            jax.ShapeDtypeStruct((N, 1), jnp.float32),
        ],
    )(emb, pc, *su_chunks, wemb, bemb, wchg, bchg,
      w1, b1, w2, b2, w3a, b3a, w4a, b4a, w3q, b3q, w4qt, b4q)


def kernel(atomic_embedding, pair_indices, f_ij_cutoff, r_ij, partial_charges,
           W_emb, b_emb, W_chg, b_chg, W1, b1, W2, b2,
           W3a, b3a, W4a, b4a, W3q, b3q, W4q, b4q):
    idx_j = pair_indices[1]
    u_t = _unit_vectors(r_ij)
    su_chunks = _segment_sums(idx_j, u_t, f_ij_cutoff)
    delta_a, delta_q = _dense_stage(
        atomic_embedding, partial_charges, su_chunks,
        W_emb, b_emb.reshape(1, F), W_chg, b_chg.reshape(1, F),
        W1, b1.reshape(1, 128), W2, b2.reshape(1, 64),
        W3a, b3a.reshape(1, 32), W4a, b4a.reshape(1, F),
        W3q, b3q.reshape(1, 32), W4q.reshape(1, 32), b4q.reshape(1, 1),
    )
    return (delta_a, delta_q)


# speedup vs baseline: 1.2950x; 1.2950x over previous
import functools

import jax
import jax.numpy as jnp
from jax import lax
from jax.experimental import pallas as pl
from jax.experimental.pallas import tpu as pltpu
from jax.experimental.pallas import tpu_sc as plsc

N, E, F = 10000, 160000, 128
FC = 32
NC = F // FC
NS = 16
EB = 80
BPT = E // EB // NS
RB = 80
NSLAB = N // RB


def _unit_body(r_ref, u_ref):
    r = r_ref[...]
    n2 = jnp.sum(r * r, axis=0, keepdims=True)
    u_ref[...] = r / jnp.sqrt(n2 + 1e-12)


def _unit_vectors(r_t):
    return pl.pallas_call(
        _unit_body,
        out_shape=jax.ShapeDtypeStruct((3, E), jnp.float32),
    )(r_t)


def _sc_body(idx_hbm, u_hbm, f_hbm, out0, out1, out2, out3,
             acc, idx0_v, idx1_v, idx2_v, u0_v, u1_v, u2_v,
             vals0_v, vals1_v, vals2_v, st_v,
             dsem0, dsem1, dsem2, ssem0, ssem1, ssem2):
    outs = (out0, out1, out2, out3)
    idx_bufs = (idx0_v, idx1_v, idx2_v)
    u_bufs = (u0_v, u1_v, u2_v)
    vals_bufs = (vals0_v, vals1_v, vals2_v)
    dsems = (dsem0, dsem1, dsem2)
    ssems = (ssem0, ssem1, ssem2)
    c = lax.axis_index("c")
    s = lax.axis_index("s")
    n_my_slabs = (NSLAB // NS) + jnp.where(s < NSLAB % NS, 1, 0)

    zero16 = jnp.zeros((16,), jnp.float32)

    def start_dma(b, p):
        e0 = (s + b * NS) * EB
        pltpu.async_copy(idx_hbm.at[pl.ds(e0, EB)], idx_bufs[p], dsems[p])
        for d in range(3):
            pltpu.async_copy(u_hbm.at[pl.ds(d * E + e0, EB)],
                             u_bufs[p].at[pl.ds(d * EB, EB)], dsems[p])
        pltpu.async_copy(f_hbm.at[pl.ds(e0, EB), :], vals_bufs[p], dsems[p])

    def wait_dma(p):
        pltpu.make_async_copy(idx_hbm.at[pl.ds(0, EB)], idx_bufs[p], dsems[p]).wait()
        for d in range(3):
            pltpu.make_async_copy(u_hbm.at[pl.ds(0, EB)],
                                  u_bufs[p].at[pl.ds(0, EB)], dsems[p]).wait()
        pltpu.make_async_copy(f_hbm.at[pl.ds(0, EB), :], vals_bufs[p], dsems[p]).wait()

    def start_scatter(p):
        pltpu.async_copy(vals_bufs[p], acc.at[idx_bufs[p]], ssems[p], add=True)

    def wait_scatter(p):
        pltpu.make_async_copy(vals_bufs[p], acc.at[idx_bufs[p]], ssems[p]).wait()

    def _pack_su(n_rows, s_off, t_offs):
        def sq_body(i, carry):
            for gg in range(FC // 16):
                x0 = vals0_v[i, pl.ds(t_offs[0] + gg * 16, 16)]
                x1 = vals0_v[i, pl.ds(t_offs[1] + gg * 16, 16)]
                x2 = vals0_v[i, pl.ds(t_offs[2] + gg * 16, 16)]
                sv = vals0_v[i, pl.ds(s_off + gg * 16, 16)]
                st_v[pl.ds(i * 64 + gg * 16, 16)] = sv
                st_v[pl.ds(i * 64 + 32 + gg * 16, 16)] = x0 * x0 + x1 * x1 + x2 * x2
            return carry

        lax.fori_loop(0, n_rows, sq_body, 0)

    for k in range(2):
        cidx = c * 2 + k
        col0 = pl.multiple_of(cidx * FC, FC)

        def compute_block(p):
            vals_v = vals_bufs[p]
            u_v = u_bufs[p]

            for core_id in range(2):
                cidx_s = core_id * 2 + k
                cs = cidx_s * FC
                ts = tuple(FC * j + (FC if j >= cidx_s else 0)
                           for j in range(3))

                @pl.when(c == core_id)
                def _(cs=cs, ts=ts):
                    def group_body(g, gcarry):
                        base = pl.multiple_of(g * 16, 16)
                        ur0 = u_v[pl.ds(base, 16)]
                        ur1 = u_v[pl.ds(EB + base, 16)]
                        ur2 = u_v[pl.ds(2 * EB + base, 16)]
                        for lane in range(16):
                            e = base + lane
                            u0 = ur0[lane]
                            u1 = ur1[lane]
                            u2 = ur2[lane]
                            for gg in range(FC // 16):
                                fr = vals_v[e, pl.ds(cs + gg * 16, 16)]
                                vals_v[e, pl.ds(ts[0] + gg * 16, 16)] = u0 * fr
                                vals_v[e, pl.ds(ts[1] + gg * 16, 16)] = u1 * fr
                                vals_v[e, pl.ds(ts[2] + gg * 16, 16)] = u2 * fr
                        return gcarry

                    lax.fori_loop(0, EB // 16, group_body, 0)

        def zfill(i, carry):
            for gg in range(F // 16):
                vals0_v[i, pl.ds(gg * 16, 16)] = zero16
            return carry

        lax.fori_loop(0, RB, zfill, 0)

        def zslab(i, carry):
            r0 = (s + i * NS) * RB
            pltpu.sync_copy(vals0_v, acc.at[pl.ds(r0, RB), :])
            return carry

        lax.fori_loop(0, n_my_slabs, zslab, 0)
        plsc.subcore_barrier()

        def slot(b, j):
            p = j % 3
            wait_dma(p)
            compute_block(p)

            @pl.when(b > 0)
            def _():
                wait_scatter((j + 2) % 3)

            start_scatter(p)

            @pl.when(b + 2 < BPT)
            def _():
                start_dma(b + 2, (j + 2) % 3)

        start_dma(0, 0)
        start_dma(1, 1)

        def tri_body(i, carry):
            for j in range(3):
                slot(3 * i + j, j)
            return carry

        lax.fori_loop(0, BPT // 3, tri_body, 0)
        for j in range(BPT - 3 * (BPT // 3)):
            slot(3 * (BPT // 3) + j, j)
        wait_scatter((BPT - 1) % 3)
        plsc.subcore_barrier()

        for core_id in range(2):
            out_ref = outs[core_id * 2 + k]
            cidx_s = core_id * 2 + k
            cs = cidx_s * FC
            ts = tuple(FC * j + (FC if j >= cidx_s else 0) for j in range(3))

            @pl.when(c == core_id)
            def _(cs=cs, ts=ts, out_ref=out_ref):
                def fslab(i, carry):
                    r0 = (s + i * NS) * RB
                    pltpu.sync_copy(acc.at[pl.ds(r0, RB), :], vals0_v)
                    _pack_su(RB, cs, ts)
                    pltpu.sync_copy(st_v, out_ref.at[pl.ds(r0 * 64, RB * 64)])
                    return carry

                lax.fori_loop(0, n_my_slabs, fslab, 0)

        plsc.subcore_barrier()


def _segment_sums(idx_j, u_t, f_ij):
    mesh = plsc.VectorSubcoreMesh(core_axis_name="c", subcore_axis_name="s")
    fn = functools.partial(
        pl.kernel,
        out_type=[jax.ShapeDtypeStruct((N * 64,), jnp.float32)] * 4,
        mesh=mesh,
        scratch_types=[
            pltpu.VMEM_SHARED((N, F), jnp.float32),
            pltpu.VMEM((EB,), jnp.int32),
            pltpu.VMEM((EB,), jnp.int32),
            pltpu.VMEM((EB,), jnp.int32),
            pltpu.VMEM((3 * EB,), jnp.float32),
            pltpu.VMEM((3 * EB,), jnp.float32),
            pltpu.VMEM((3 * EB,), jnp.float32),
            pltpu.VMEM((EB, F), jnp.float32),
            pltpu.VMEM((EB, F), jnp.float32),
            pltpu.VMEM((EB, F), jnp.float32),
            pltpu.VMEM((RB * 64,), jnp.float32),
            pltpu.SemaphoreType.DMA,
            pltpu.SemaphoreType.DMA,
            pltpu.SemaphoreType.DMA,
            pltpu.SemaphoreType.DMA,
            pltpu.SemaphoreType.DMA,
            pltpu.SemaphoreType.DMA,
        ],
    )(_sc_body)
    return [o.reshape(N, 64) for o in fn(idx_j, u_t.reshape(3 * E), f_ij)]


def _gelu(x):
    return jax.nn.gelu(x)


def _dense_body(emb_ref, pc_ref, o0_ref, o1_ref, o2_ref, o3_ref,
                wemb_ref, bemb_ref, wchg_ref, bchg_ref,
                w1_ref, b1_ref, w2_ref, b2_ref,
                w3a_ref, b3a_ref, w4a_ref, b4a_ref,
                w3q_ref, b3q_ref, w4qt_ref, b4q_ref,
                da_ref, dq_ref):
    emb = emb_ref[...]
    a_t = jnp.dot(emb, wemb_ref[...], preferred_element_type=jnp.float32) + bemb_ref[...]
    q_t = pc_ref[...] * jnp.sum(wchg_ref[...], axis=0, keepdims=True) + bchg_ref[...]

    o_blocks = [o_ref[...] for o_ref in (o0_ref, o1_ref, o2_ref, o3_ref)]
    s_v = jnp.concatenate([o[:, :FC] for o in o_blocks], axis=1)
    uu = jnp.concatenate([o[:, FC:] for o in o_blocks], axis=1)
    radial_a = a_t * s_v
    vector_a = jnp.sqrt(a_t * a_t * uu + 1e-12)
    radial_q = q_t * s_v
    vector_q = jnp.sqrt(q_t * q_t * uu + 1e-12)
    comb = jnp.concatenate([radial_a, vector_a, radial_q, vector_q], axis=1)

    h = _gelu(jnp.dot(comb, w1_ref[...], preferred_element_type=jnp.float32) + b1_ref[...])
    h = _gelu(jnp.dot(h, w2_ref[...], preferred_element_type=jnp.float32) + b2_ref[...])
    ha = _gelu(jnp.dot(h, w3a_ref[...], preferred_element_type=jnp.float32) + b3a_ref[...])
    da_ref[...] = jnp.dot(ha, w4a_ref[...], preferred_element_type=jnp.float32) + b4a_ref[...]
    hq = _gelu(jnp.dot(h, w3q_ref[...], preferred_element_type=jnp.float32) + b3q_ref[...])
    dq_ref[...] = jnp.sum(hq * w4qt_ref[...], axis=1, keepdims=True) + b4q_ref[...]


def _dense_stage(emb, pc, su_chunks, wemb, bemb, wchg, bchg,
                 w1, b1, w2, b2, w3a, b3a, w4a, b4a, w3q, b3q, w4qt, b4q):
    nb = 1000
    grid = N // nb
    row_spec = lambda shape: pl.BlockSpec((nb,) + shape[1:], lambda i: (i,) + (0,) * (len(shape) - 1))
    full_spec = lambda shape: pl.BlockSpec(shape, lambda i: (0,) * len(shape))
    in_specs = [
        row_spec((N, F)), row_spec((N, 1)),
        row_spec((N, 64)), row_spec((N, 64)), row_spec((N, 64)), row_spec((N, 64)),
        full_spec(wemb.shape), full_spec(bemb.shape),
        full_spec(wchg.shape), full_spec(bchg.shape),
        full_spec(w1.shape), full_spec(b1.shape),
        full_spec(w2.shape), full_spec(b2.shape),
        full_spec(w3a.shape), full_spec(b3a.shape),
        full_spec(w4a.shape), full_spec(b4a.shape),
        full_spec(w3q.shape), full_spec(b3q.shape),
        full_spec(w4qt.shape), full_spec(b4q.shape),
    ]
    return pl.pallas_call(
        _dense_body,
        grid=(grid,),
        in_specs=in_specs,
        out_specs=[
            pl.BlockSpec((nb, F), lambda i: (i, 0)),
            pl.BlockSpec((nb, 1), lambda i: (i, 0)),
        ],
        out_shape=[
            jax.ShapeDtypeStruct((N, F), jnp.float32),
            jax.ShapeDtypeStruct((N, 1), jnp.float32),
        ],
    )(emb, pc, *su_chunks, wemb, bemb, wchg, bchg,
      w1, b1, w2, b2, w3a, b3a, w4a, b4a, w3q, b3q, w4qt, b4q)


def kernel(atomic_embedding, pair_indices, f_ij_cutoff, r_ij, partial_charges,
           W_emb, b_emb, W_chg, b_chg, W1, b1, W2, b2,
           W3a, b3a, W4a, b4a, W3q, b3q, W4q, b4q):
    idx_j = pair_indices[1]
    u_t = _unit_vectors(r_ij.T)
    su_chunks = _segment_sums(idx_j, u_t, f_ij_cutoff)
    delta_a, delta_q = _dense_stage(
        atomic_embedding, partial_charges, su_chunks,
        W_emb, b_emb.reshape(1, F), W_chg, b_chg.reshape(1, F),
        W1, b1.reshape(1, 128), W2, b2.reshape(1, 64),
        W3a, b3a.reshape(1, 32), W4a, b4a.reshape(1, F),
        W3q, b3q.reshape(1, 32), W4q.reshape(1, 32), b4q.reshape(1, 1),
    )
    return (delta_a, delta_q)
